# baseline (speedup 1.0000x reference)
import jax
import jax.numpy as jnp
from jax import lax
from jax.experimental import pallas as pl
from jax.experimental.pallas import tpu as pltpu

NUM_RIMS = 16
NUM_HEADS = 4
DEPTH = 32
C = 192
TOP_K = 4
G = NUM_RIMS * NUM_HEADS
QKV = G * DEPTH

BN = 2048
RPAD = 128


def _fused_body(x_ref, kvt_ref, kvb_ref, qbd_ref, kb_ref, mbig_ref,
                mb_ref, ot_ref, ob_ref, o_ref):
    xb = x_ref[...]
    kv = jnp.maximum(
        jnp.dot(xb, kvt_ref[...], preferred_element_type=jnp.float32)
        + kvb_ref[...], 0.0)
    k = kv[:, :QKV]
    v = kv[:, QKV:]
    a = jnp.dot(k, qbd_ref[...], preferred_element_type=jnp.float32)
    cz = jnp.dot(kb_ref[...], qbd_ref[...],
                 preferred_element_type=jnp.float32)
    m = jnp.maximum(a, cz)
    e0 = jnp.exp(a - m)
    e1 = jnp.exp(cz - m)
    attn0 = e0 / (e0 + e1)
    s16 = ((attn0[:, 0:NUM_RIMS] + attn0[:, NUM_RIMS:2 * NUM_RIMS])
           + attn0[:, 2 * NUM_RIMS:3 * NUM_RIMS]) \
        + attn0[:, 3 * NUM_RIMS:4 * NUM_RIMS]
    s = s16
    iota = lax.broadcasted_iota(jnp.int32, (BN, NUM_RIMS), 1)
    mask = jnp.zeros((BN, NUM_RIMS), jnp.float32)
    for _ in range(TOP_K):
        mx = jnp.max(s, axis=1, keepdims=True)
        first = jnp.min(jnp.where(s == mx, iota, NUM_RIMS), axis=1,
                        keepdims=True)
        sel = iota == first
        mask = mask + sel.astype(jnp.float32)
        s = jnp.where(sel, -1e30, s)
    mf = jnp.reshape(
        jnp.broadcast_to(mask[:, :, None], (BN, NUM_RIMS, QKV // NUM_RIMS)),
        (BN, QKV))
    pre = (jnp.dot(v * mf, mbig_ref[...], preferred_element_type=jnp.float32)
           + jnp.dot(mask, mb_ref[...], preferred_element_type=jnp.float32)
           ) * (1.0 / TOP_K)
    o = jnp.maximum(pre, 0.0)
    o_ref[...] = jnp.maximum(
        jnp.dot(o, ot_ref[...], preferred_element_type=jnp.float32)
        + ob_ref[...], 0.0)


@jax.jit
def kernel(x, rims, key_W, key_b, value_W, value_b, query_W, query_b,
           merge_W, merge_b, out_W, out_b):
    B, _, H, W = x.shape
    N = B * H * W

    kvt = jnp.concatenate([key_W.T, value_W.T], axis=1)
    kvb = jnp.concatenate([key_b, value_b])[None, :]
    wq = query_W.reshape(NUM_RIMS, QKV // NUM_RIMS, C)
    rims_r = rims.reshape(NUM_RIMS, C)
    q = jax.nn.relu(
        jnp.einsum('rij,rj->ri', wq, rims_r)
        + query_b.reshape(NUM_RIMS, QKV // NUM_RIMS)).reshape(G, DEPTH)
    qbd = (q[:, :, None] * jnp.eye(G, dtype=jnp.float32)[:, None, :]
           ).reshape(QKV, G)
    perm = (jnp.arange(G) % NUM_RIMS) * NUM_HEADS \
        + (jnp.arange(G) // NUM_RIMS)
    qbd = qbd[:, perm]
    kbrow = jax.nn.relu(key_b)[None, :]
    mbig = merge_W.reshape(NUM_RIMS, C, QKV // NUM_RIMS) \
        .transpose(0, 2, 1).reshape(QKV, C)
    mb = merge_b.reshape(NUM_RIMS, C)
    ot = out_W.T
    ob = out_b[None, :]

    xr = x.transpose(0, 2, 3, 1).reshape(N, C)

    full = lambda shape: pl.BlockSpec(shape, lambda i: (0, 0))
    out = pl.pallas_call(
        _fused_body,
        grid=(N // BN,),
        in_specs=[
            pl.BlockSpec((BN, C), lambda i: (i, 0)),
            full((C, 2 * QKV)),
            full((1, 2 * QKV)),
            full((QKV, G)),
            full((1, QKV)),
            full((QKV, C)),
            full((NUM_RIMS, C)),
            full((C, C)),
            full((1, C)),
        ],
        out_specs=pl.BlockSpec((BN, C), lambda i: (i, 0)),
        out_shape=jax.ShapeDtypeStruct((N, C), jnp.float32),
        compiler_params=pltpu.CompilerParams(
            dimension_semantics=("parallel",),
        ),
    )(xr, kvt, kvb, qbd, kbrow, mbig, mb, ot, ob)

    return out.reshape(B, H, W, C).transpose(0, 3, 1, 2)


# speedup vs baseline: 1.4865x; 1.4865x over previous
import jax
import jax.numpy as jnp
from jax import lax
from jax.experimental import pallas as pl
from jax.experimental.pallas import tpu as pltpu

NUM_RIMS = 16
NUM_HEADS = 4
DEPTH = 32
C = 192
TOP_K = 4
G = NUM_RIMS * NUM_HEADS
QKV = G * DEPTH

BN = 2048
RPAD = 128


def _fused_body(x_ref, kvt_ref, kvb_ref, qbd_ref, kb_ref, e_ref, mbig_ref,
                mb_ref, ot_ref, ob_ref, o_ref):
    xb = x_ref[...]
    kv = jnp.maximum(
        jnp.dot(xb, kvt_ref[...], preferred_element_type=jnp.float32)
        + kvb_ref[...], 0.0)
    k = kv[:, :QKV]
    v = kv[:, QKV:]
    a = jnp.dot(k, qbd_ref[...], preferred_element_type=jnp.float32)
    cz = jnp.dot(kb_ref[...], qbd_ref[...],
                 preferred_element_type=jnp.float32)
    m = jnp.maximum(a, cz)
    e0 = jnp.exp(a - m)
    e1 = jnp.exp(cz - m)
    attn0 = e0 / (e0 + e1)
    s16 = ((attn0[:, 0:NUM_RIMS] + attn0[:, NUM_RIMS:2 * NUM_RIMS])
           + attn0[:, 2 * NUM_RIMS:3 * NUM_RIMS]) \
        + attn0[:, 3 * NUM_RIMS:4 * NUM_RIMS]
    s = jnp.concatenate(
        [s16, jnp.full((BN, RPAD - NUM_RIMS), -1e30, jnp.float32)], axis=1)
    iota = lax.broadcasted_iota(jnp.int32, (BN, RPAD), 1)
    mask = jnp.zeros((BN, RPAD), jnp.float32)
    for _ in range(TOP_K):
        mx = jnp.max(s, axis=1, keepdims=True)
        first = jnp.min(jnp.where(s == mx, iota, RPAD), axis=1,
                        keepdims=True)
        sel = iota == first
        mask = mask + sel.astype(jnp.float32)
        s = jnp.where(sel, -1e30, s)
    mask16 = mask[:, :NUM_RIMS]
    mf = jnp.dot(mask16, e_ref[...], preferred_element_type=jnp.float32)
    pre = (jnp.dot(v * mf, mbig_ref[...], preferred_element_type=jnp.float32)
           + jnp.dot(mask16, mb_ref[...], preferred_element_type=jnp.float32)
           ) * (1.0 / TOP_K)
    o = jnp.maximum(pre, 0.0)
    o_ref[...] = jnp.maximum(
        jnp.dot(o, ot_ref[...], preferred_element_type=jnp.float32)
        + ob_ref[...], 0.0)


@jax.jit
def kernel(x, rims, key_W, key_b, value_W, value_b, query_W, query_b,
           merge_W, merge_b, out_W, out_b):
    B, _, H, W = x.shape
    N = B * H * W

    kvt = jnp.concatenate([key_W.T, value_W.T], axis=1)
    kvb = jnp.concatenate([key_b, value_b])[None, :]
    wq = query_W.reshape(NUM_RIMS, QKV // NUM_RIMS, C)
    rims_r = rims.reshape(NUM_RIMS, C)
    q = jax.nn.relu(
        jnp.einsum('rij,rj->ri', wq, rims_r)
        + query_b.reshape(NUM_RIMS, QKV // NUM_RIMS)).reshape(G, DEPTH)
    qbd = (q[:, :, None] * jnp.eye(G, dtype=jnp.float32)[:, None, :]
           ).reshape(QKV, G)
    perm = (jnp.arange(G) % NUM_RIMS) * NUM_HEADS \
        + (jnp.arange(G) // NUM_RIMS)
    qbd = qbd[:, perm]
    kbrow = jax.nn.relu(key_b)[None, :]
    e = jnp.repeat(jnp.eye(NUM_RIMS, dtype=jnp.float32),
                   QKV // NUM_RIMS, axis=1)
    mbig = merge_W.reshape(NUM_RIMS, C, QKV // NUM_RIMS) \
        .transpose(0, 2, 1).reshape(QKV, C)
    mb = merge_b.reshape(NUM_RIMS, C)
    ot = out_W.T
    ob = out_b[None, :]

    xr = x.transpose(0, 2, 3, 1).reshape(N, C)

    full = lambda shape: pl.BlockSpec(shape, lambda i: (0, 0))
    out = pl.pallas_call(
        _fused_body,
        grid=(N // BN,),
        in_specs=[
            pl.BlockSpec((BN, C), lambda i: (i, 0)),
            full((C, 2 * QKV)),
            full((1, 2 * QKV)),
            full((QKV, G)),
            full((1, QKV)),
            full((NUM_RIMS, QKV)),
            full((QKV, C)),
            full((NUM_RIMS, C)),
            full((C, C)),
            full((1, C)),
        ],
        out_specs=pl.BlockSpec((BN, C), lambda i: (i, 0)),
        out_shape=jax.ShapeDtypeStruct((N, C), jnp.float32),
        compiler_params=pltpu.CompilerParams(
            dimension_semantics=("parallel",),
        ),
    )(xr, kvt, kvb, qbd, kbrow, e, mbig, mb, ot, ob)

    return out.reshape(B, H, W, C).transpose(0, 3, 1, 2)


# speedup vs baseline: 1.4911x; 1.0031x over previous
import jax
import jax.numpy as jnp
from jax import lax
from jax.experimental import pallas as pl
from jax.experimental.pallas import tpu as pltpu

NUM_RIMS = 16
NUM_HEADS = 4
DEPTH = 32
C = 192
TOP_K = 4
G = NUM_RIMS * NUM_HEADS
QKV = G * DEPTH

BN = 2048
RPAD = 128


def _fused_body(x_ref, kvt_ref, kvb_ref, qbd_ref, kb_ref, e_ref, mbig_ref,
                mb_ref, ot_ref, ob_ref, o_ref):
    xb = x_ref[...]
    kv = jnp.maximum(
        jnp.dot(xb, kvt_ref[...], preferred_element_type=jnp.float32)
        + kvb_ref[...], 0.0)
    k = kv[:, :QKV]
    v = kv[:, QKV:]
    a = jnp.dot(k, qbd_ref[...], preferred_element_type=jnp.float32)
    cz = jnp.dot(kb_ref[...], qbd_ref[...],
                 preferred_element_type=jnp.float32)
    m = jnp.maximum(a, cz)
    e0 = jnp.exp(a - m)
    e1 = jnp.exp(cz - m)
    attn0 = e0 / (e0 + e1)
    s16 = ((attn0[:, 0:NUM_RIMS] + attn0[:, NUM_RIMS:2 * NUM_RIMS])
           + attn0[:, 2 * NUM_RIMS:3 * NUM_RIMS]) \
        + attn0[:, 3 * NUM_RIMS:4 * NUM_RIMS]
    s = s16
    iota = lax.broadcasted_iota(jnp.int32, (BN, NUM_RIMS), 1)
    mask16 = jnp.zeros((BN, NUM_RIMS), jnp.float32)
    for _ in range(TOP_K):
        mx = jnp.max(s, axis=1, keepdims=True)
        first = jnp.min(jnp.where(s == mx, iota, NUM_RIMS), axis=1,
                        keepdims=True)
        sel = iota == first
        mask16 = mask16 + sel.astype(jnp.float32)
        s = jnp.where(sel, -1e30, s)
    mf = jnp.dot(mask16, e_ref[...], preferred_element_type=jnp.float32)
    pre = (jnp.dot(v * mf, mbig_ref[...], preferred_element_type=jnp.float32)
           + jnp.dot(mask16, mb_ref[...], preferred_element_type=jnp.float32)
           ) * (1.0 / TOP_K)
    o = jnp.maximum(pre, 0.0)
    o_ref[...] = jnp.maximum(
        jnp.dot(o, ot_ref[...], preferred_element_type=jnp.float32)
        + ob_ref[...], 0.0)


@jax.jit
def kernel(x, rims, key_W, key_b, value_W, value_b, query_W, query_b,
           merge_W, merge_b, out_W, out_b):
    B, _, H, W = x.shape
    N = B * H * W

    kvt = jnp.concatenate([key_W.T, value_W.T], axis=1)
    kvb = jnp.concatenate([key_b, value_b])[None, :]
    wq = query_W.reshape(NUM_RIMS, QKV // NUM_RIMS, C)
    rims_r = rims.reshape(NUM_RIMS, C)
    q = jax.nn.relu(
        jnp.einsum('rij,rj->ri', wq, rims_r)
        + query_b.reshape(NUM_RIMS, QKV // NUM_RIMS)).reshape(G, DEPTH)
    qbd = (q[:, :, None] * jnp.eye(G, dtype=jnp.float32)[:, None, :]
           ).reshape(QKV, G)
    perm = (jnp.arange(G) % NUM_RIMS) * NUM_HEADS \
        + (jnp.arange(G) // NUM_RIMS)
    qbd = qbd[:, perm]
    kbrow = jax.nn.relu(key_b)[None, :]
    e = jnp.repeat(jnp.eye(NUM_RIMS, dtype=jnp.float32),
                   QKV // NUM_RIMS, axis=1)
    mbig = merge_W.reshape(NUM_RIMS, C, QKV // NUM_RIMS) \
        .transpose(0, 2, 1).reshape(QKV, C)
    mb = merge_b.reshape(NUM_RIMS, C)
    ot = out_W.T
    ob = out_b[None, :]

    xr = x.transpose(0, 2, 3, 1).reshape(N, C)

    full = lambda shape: pl.BlockSpec(shape, lambda i: (0, 0))
    out = pl.pallas_call(
        _fused_body,
        grid=(N // BN,),
        in_specs=[
            pl.BlockSpec((BN, C), lambda i: (i, 0)),
            full((C, 2 * QKV)),
            full((1, 2 * QKV)),
            full((QKV, G)),
            full((1, QKV)),
            full((NUM_RIMS, QKV)),
            full((QKV, C)),
            full((NUM_RIMS, C)),
            full((C, C)),
            full((1, C)),
        ],
        out_specs=pl.BlockSpec((BN, C), lambda i: (i, 0)),
        out_shape=jax.ShapeDtypeStruct((N, C), jnp.float32),
        compiler_params=pltpu.CompilerParams(
            dimension_semantics=("parallel",),
        ),
    )(xr, kvt, kvb, qbd, kbrow, e, mbig, mb, ot, ob)

    return out.reshape(B, H, W, C).transpose(0, 3, 1, 2)
